# Initial kernel scaffold; baseline (speedup 1.0000x reference)
import jax
import jax.numpy as jnp
from jax.experimental import pallas as pl


def kernel(input1, input2):
    raise NotImplementedError("write your pallas kernel here")


# speedup vs baseline: 2.1558x; 2.1558x over previous
import jax
import jax.numpy as jnp
from jax.experimental import pallas as pl
from jax.experimental.pallas import tpu as pltpu


def _chamfer_body(nc, n, m, x1_ref, x2t_ref, d1_ref, i1_ref, d2_ref, i2_ref):
    i = pl.program_id(1)
    x1 = x1_ref[0]
    x2t = x2t_ref[0]

    sq1 = jnp.sum(x1 * x1, axis=1, keepdims=True)
    sq2 = jnp.sum(x2t * x2t, axis=0, keepdims=True)
    inner = jax.lax.dot_general(
        x1.astype(jnp.bfloat16), x2t.astype(jnp.bfloat16),
        (((1,), (0,)), ((), ())),
        preferred_element_type=jnp.float32)
    d = sq1 + sq2 - 2.0 * inner

    rmin = jnp.min(d, axis=1, keepdims=True)
    col_iota = jax.lax.broadcasted_iota(jnp.int32, d.shape, 1)
    rarg = jnp.min(jnp.where(d == rmin, col_iota, m),
                   axis=1, keepdims=True)
    d1_ref[0] = rmin
    i1_ref[0] = rarg

    cmin = jnp.min(d, axis=0, keepdims=True)
    row_iota = jax.lax.broadcasted_iota(jnp.int32, d.shape, 0) + i * nc
    carg = jnp.min(jnp.where(d == cmin, row_iota, n),
                   axis=0, keepdims=True)

    @pl.when(i == 0)
    def _():
        d2_ref[0] = cmin
        i2_ref[0] = carg

    @pl.when(i != 0)
    def _():
        prev_d = d2_ref[0]
        prev_i = i2_ref[0]
        better = cmin < prev_d
        d2_ref[0] = jnp.where(better, cmin, prev_d)
        i2_ref[0] = jnp.where(better, carg, prev_i)


def _chamfer_onedir(x1, x2t, nc):
    b, n, _ = x1.shape
    m = x2t.shape[2]
    n_chunks = n // nc

    def body(x1_ref, x2t_ref, d1_ref, i1_ref, d2_ref, i2_ref):
        _chamfer_body(nc, n, m, x1_ref, x2t_ref, d1_ref, i1_ref, d2_ref,
                      i2_ref)

    d1, i1, d2, i2 = pl.pallas_call(
        body,
        grid=(b, n_chunks),
        in_specs=[
            pl.BlockSpec((1, nc, 3), lambda bb, ii: (bb, ii, 0)),
            pl.BlockSpec((1, 3, m), lambda bb, ii: (bb, 0, 0)),
        ],
        out_specs=[
            pl.BlockSpec((1, nc, 1), lambda bb, ii: (bb, ii, 0)),
            pl.BlockSpec((1, nc, 1), lambda bb, ii: (bb, ii, 0)),
            pl.BlockSpec((1, 1, m), lambda bb, ii: (bb, 0, 0)),
            pl.BlockSpec((1, 1, m), lambda bb, ii: (bb, 0, 0)),
        ],
        out_shape=[
            jax.ShapeDtypeStruct((b, n, 1), jnp.float32),
            jax.ShapeDtypeStruct((b, n, 1), jnp.int32),
            jax.ShapeDtypeStruct((b, 1, m), jnp.float32),
            jax.ShapeDtypeStruct((b, 1, m), jnp.int32),
        ],
        compiler_params=pltpu.CompilerParams(
            dimension_semantics=("parallel", "arbitrary")),
    )(x1, x2t)
    return (d1.reshape(b, n), i1.reshape(b, n),
            d2.reshape(b, m), i2.reshape(b, m))


@jax.jit
def kernel(input1, input2):
    x2t = input2.transpose(0, 2, 1)
    dist1, idx1, dist2, idx2 = _chamfer_onedir(input1, x2t, 512)
    return (dist1, dist2, idx1, idx2)


# speedup vs baseline: 2.5407x; 1.1785x over previous
import jax
import jax.numpy as jnp
from jax.experimental import pallas as pl
from jax.experimental.pallas import tpu as pltpu


def _chamfer_body(nc, n, m, x1_ref, x2t_ref, d1_ref, i1_ref, d2_ref, i2_ref):
    i = pl.program_id(1)
    x1 = x1_ref[0]
    x2t = x2t_ref[0]

    sq1 = jnp.sum(x1 * x1, axis=1, keepdims=True)
    sq2 = jnp.sum(x2t * x2t, axis=0, keepdims=True)
    inner2 = jax.lax.dot_general(
        x1.astype(jnp.bfloat16), (x2t * -2.0).astype(jnp.bfloat16),
        (((1,), (0,)), ((), ())),
        preferred_element_type=jnp.float32)
    d = (sq1 + sq2) + inner2

    rmin = jnp.min(d, axis=1, keepdims=True)
    rarg = jnp.argmin(d, axis=1).astype(jnp.int32)[:, None]
    d1_ref[0] = rmin
    i1_ref[0] = rarg

    cmin = jnp.min(d, axis=0, keepdims=True)
    carg = jnp.argmin(d, axis=0).astype(jnp.int32)[None, :] + i * nc

    @pl.when(i == 0)
    def _():
        d2_ref[0] = cmin
        i2_ref[0] = carg

    @pl.when(i != 0)
    def _():
        prev_d = d2_ref[0]
        prev_i = i2_ref[0]
        better = cmin < prev_d
        d2_ref[0] = jnp.where(better, cmin, prev_d)
        i2_ref[0] = jnp.where(better, carg, prev_i)


def _chamfer_onedir(x1, x2t, nc):
    b, n, _ = x1.shape
    m = x2t.shape[2]
    n_chunks = n // nc

    def body(x1_ref, x2t_ref, d1_ref, i1_ref, d2_ref, i2_ref):
        _chamfer_body(nc, n, m, x1_ref, x2t_ref, d1_ref, i1_ref, d2_ref,
                      i2_ref)

    d1, i1, d2, i2 = pl.pallas_call(
        body,
        grid=(b, n_chunks),
        in_specs=[
            pl.BlockSpec((1, nc, 3), lambda bb, ii: (bb, ii, 0)),
            pl.BlockSpec((1, 3, m), lambda bb, ii: (bb, 0, 0)),
        ],
        out_specs=[
            pl.BlockSpec((1, nc, 1), lambda bb, ii: (bb, ii, 0)),
            pl.BlockSpec((1, nc, 1), lambda bb, ii: (bb, ii, 0)),
            pl.BlockSpec((1, 1, m), lambda bb, ii: (bb, 0, 0)),
            pl.BlockSpec((1, 1, m), lambda bb, ii: (bb, 0, 0)),
        ],
        out_shape=[
            jax.ShapeDtypeStruct((b, n, 1), jnp.float32),
            jax.ShapeDtypeStruct((b, n, 1), jnp.int32),
            jax.ShapeDtypeStruct((b, 1, m), jnp.float32),
            jax.ShapeDtypeStruct((b, 1, m), jnp.int32),
        ],
        compiler_params=pltpu.CompilerParams(
            dimension_semantics=("parallel", "arbitrary")),
    )(x1, x2t)
    return (d1.reshape(b, n), i1.reshape(b, n),
            d2.reshape(b, m), i2.reshape(b, m))


@jax.jit
def kernel(input1, input2):
    x2t = input2.transpose(0, 2, 1)
    dist1, idx1, dist2, idx2 = _chamfer_onedir(input1, x2t, 1024)
    return (dist1, dist2, idx1, idx2)
